# baseline (speedup 1.0000x reference)
import functools

import jax
import jax.numpy as jnp
from jax import lax
from jax.experimental import pallas as pl
from jax.experimental.pallas import tpu as pltpu
from jax.experimental.pallas import tpu_sc as plsc

_BATCH = 16384
_HIDDEN = 64
_LMBDA = 0.01
_NC = 2
_NS = 16
_NW = _NC * _NS
_PER_W = _BATCH // _NW
_CH = 16
_NCHK = _PER_W // _CH
_L = 16
_NCHD = _HIDDEN // _L


def _chunk_scores(gh, gr, gt, lane, sq_ref, sb, off):
    sq_acc = None
    acc = jnp.zeros((_L,), jnp.float32)
    for i in range(_CH):
        hs = [gh[i, pl.ds(_L * c, _L)] for c in range(_NCHD)]
        rs = [gr[i, pl.ds(_L * c, _L)] for c in range(_NCHD)]
        ts = [gt[i, pl.ds(_L * c, _L)] for c in range(_NCHD)]
        prods = [hs[c] * rs[c] * ts[c] for c in range(_NCHD)]
        v = (prods[0] + prods[1]) + (prods[2] + prods[3])
        acc = jnp.where(lane == i, jnp.sum(v), acc)
        s = None
        for x in hs + rs + ts:
            xx = x * x
            s = xx if s is None else s + xx
        sq_acc = s if sq_acc is None else sq_acc + s
    sq_ref[...] = sq_ref[...] + sq_acc
    sb[pl.ds(off, _L)] = acc


def _sc_body(ent_hbm, rel_hbm, ph_hbm, pt_hbm, pr_hbm, nh_hbm, nt_hbm, nr_hbm,
             ps_out, ns_out, reg_out,
             iph, ipt, ipr, inh, inT, inr,
             gph, gpt, gpr, gnh, gnt, gnr,
             sb_p, sb_n, sq_ref, sem):
    wid = lax.axis_index("s") * _NC + lax.axis_index("c")
    base = wid * _PER_W
    lane = lax.iota(jnp.int32, _L)
    sq_ref[...] = jnp.zeros((_L,), jnp.float32)

    pltpu.sync_copy(ph_hbm.at[pl.ds(base, _PER_W)], iph)
    pltpu.sync_copy(pt_hbm.at[pl.ds(base, _PER_W)], ipt)
    pltpu.sync_copy(pr_hbm.at[pl.ds(base, _PER_W)], ipr)
    pltpu.sync_copy(nh_hbm.at[pl.ds(base, _PER_W)], inh)
    pltpu.sync_copy(nt_hbm.at[pl.ds(base, _PER_W)], inT)
    pltpu.sync_copy(nr_hbm.at[pl.ds(base, _PER_W)], inr)

    def chunk_body(c, carry):
        off = c * _CH
        sl = pl.ds(off, _CH)
        qph = iph[sl]
        qpt = ipt[sl]
        qpr = ipr[sl]
        qnh = inh[sl]
        qnt = inT[sl]
        qnr = inr[sl]
        cps = []
        for i in range(_CH):
            cps.append(pltpu.async_copy(ent_hbm.at[qph[i]], gph.at[i], sem))
            cps.append(pltpu.async_copy(ent_hbm.at[qpt[i]], gpt.at[i], sem))
            cps.append(pltpu.async_copy(rel_hbm.at[qpr[i]], gpr.at[i], sem))
            cps.append(pltpu.async_copy(ent_hbm.at[qnh[i]], gnh.at[i], sem))
            cps.append(pltpu.async_copy(ent_hbm.at[qnt[i]], gnt.at[i], sem))
            cps.append(pltpu.async_copy(rel_hbm.at[qnr[i]], gnr.at[i], sem))
        for cp in cps:
            cp.wait()
        _chunk_scores(gph, gpr, gpt, lane, sq_ref, sb_p, off)
        _chunk_scores(gnh, gnr, gnt, lane, sq_ref, sb_n, off)
        return carry

    lax.fori_loop(0, _NCHK, chunk_body, 0, unroll=False)
    pltpu.sync_copy(sb_p, ps_out.at[pl.ds(base, _PER_W)])
    pltpu.sync_copy(sb_n, ns_out.at[pl.ds(base, _PER_W)])
    pltpu.sync_copy(sq_ref, reg_out.at[wid])


def _make_sc_call():
    mesh = plsc.VectorSubcoreMesh(core_axis_name="c", subcore_axis_name="s",
                                  num_cores=_NC, num_subcores=_NS)
    return pl.kernel(
        _sc_body,
        out_type=(
            jax.ShapeDtypeStruct((_BATCH,), jnp.float32),
            jax.ShapeDtypeStruct((_BATCH,), jnp.float32),
            jax.ShapeDtypeStruct((_NW, _L), jnp.float32),
        ),
        mesh=mesh,
        compiler_params=pltpu.CompilerParams(needs_layout_passes=False),
        scratch_types=(
            [pltpu.VMEM((_PER_W,), jnp.int32) for _ in range(6)]
            + [pltpu.VMEM((_CH, _HIDDEN), jnp.float32) for _ in range(6)]
            + [pltpu.VMEM((_PER_W,), jnp.float32),
               pltpu.VMEM((_PER_W,), jnp.float32),
               pltpu.VMEM((_L,), jnp.float32),
               pltpu.SemaphoreType.DMA]
        ),
    )


def _loss_body(ps_ref, ns_ref, py_ref, ny_ref, reg_ref, out_ref):
    xp = -py_ref[...] * ps_ref[...]
    xn = -ny_ref[...] * ns_ref[...]
    sp = jnp.maximum(xp, 0.0) + jnp.log(1.0 + jnp.exp(-jnp.abs(xp)))
    sn = jnp.maximum(xn, 0.0) + jnp.log(1.0 + jnp.exp(-jnp.abs(xn)))
    loss_f = (jnp.sum(sp) + jnp.sum(sn)) / _BATCH
    reg = jnp.sum(reg_ref[...]) / (_BATCH * _HIDDEN)
    out_ref[...] = jnp.zeros((1, 1), jnp.float32) + (loss_f + _LMBDA * reg)


def kernel(ent_embeddings, rel_embeddings, pos_h, pos_t, pos_r,
           neg_h, neg_t, neg_r, pos_y, neg_y):
    sc = _make_sc_call()
    ps, ns, reg = sc(ent_embeddings, rel_embeddings,
                     pos_h.astype(jnp.int32), pos_t.astype(jnp.int32),
                     pos_r.astype(jnp.int32), neg_h.astype(jnp.int32),
                     neg_t.astype(jnp.int32), neg_r.astype(jnp.int32))
    out = pl.pallas_call(
        _loss_body,
        out_shape=jax.ShapeDtypeStruct((1, 1), jnp.float32),
    )(ps.reshape(128, 128), ns.reshape(128, 128),
      pos_y.reshape(128, 128), neg_y.reshape(128, 128), reg)
    return out[0, 0]
